# baseline (speedup 1.0000x reference)
import functools

import jax
import jax.numpy as jnp
from jax import lax
from jax.experimental import pallas as pl
from jax.experimental.pallas import tpu as pltpu
from jax.experimental.pallas import tpu_sc as plsc

_B, _S, _IN_F, _OUT_F, _BANK, _K = 4, 2048, 1024, 1024, 16, 2

_NC, _NS, _L = 2, 16, 16
_NW = _NC * _NS
_RPW = (_B * _IN_F) // _NW
_CHUNK = 32
_NCH = _RPW // _CHUNK

_sc_mesh = plsc.VectorSubcoreMesh(core_axis_name="c", subcore_axis_name="s")


def _sc_blend(idx0_hbm, idx1_hbm, p0_hbm, p1_hbm, w_hbm, out_hbm,
              p0_v, p1_v, idx0_v, idx1_v, r0_v, r1_v, o_v, sem0, sem1):
    cid = lax.axis_index("c")
    sid = lax.axis_index("s")
    wid = sid * _NC + cid
    rbase = wid * _RPW
    b = wid // (_NW // _B)

    pltpu.sync_copy(p0_hbm.at[pl.ds(b * _L, _L)], p0_v)
    pltpu.sync_copy(p1_hbm.at[pl.ds(b * _L, _L)], p1_v)
    p0 = p0_v[...]
    p1 = p1_v[...]

    for c in range(_NCH):
        row = rbase + c * _CHUNK
        pltpu.sync_copy(idx0_hbm.at[pl.ds(row, _CHUNK)], idx0_v)
        pltpu.sync_copy(idx1_hbm.at[pl.ds(row, _CHUNK)], idx1_v)
        cp0 = pltpu.async_copy(w_hbm.at[idx0_v], r0_v, sem0)
        cp1 = pltpu.async_copy(w_hbm.at[idx1_v], r1_v, sem1)
        cp0.wait()
        cp1.wait()

        def _fma_row(i, carry):
            for g in range(_OUT_F // _L):
                s = pl.ds(g * _L, _L)
                o_v[i, s] = p0 * r0_v[i, s] + p1 * r1_v[i, s]
            return carry

        lax.fori_loop(0, _CHUNK, _fma_row, 0)
        pltpu.sync_copy(o_v, out_hbm.at[pl.ds(row, _CHUNK)])


_blend_call = functools.partial(
    pl.kernel,
    mesh=_sc_mesh,
    out_type=jax.ShapeDtypeStruct((_B * _IN_F, _OUT_F), jnp.float32),
    scratch_types=[
        pltpu.VMEM((_L,), jnp.float32),
        pltpu.VMEM((_L,), jnp.float32),
        pltpu.VMEM((_CHUNK,), jnp.int32),
        pltpu.VMEM((_CHUNK,), jnp.int32),
        pltpu.VMEM((_CHUNK, _OUT_F), jnp.float32),
        pltpu.VMEM((_CHUNK, _OUT_F), jnp.float32),
        pltpu.VMEM((_CHUNK, _OUT_F), jnp.float32),
        pltpu.SemaphoreType.DMA,
        pltpu.SemaphoreType.DMA,
    ],
)(_sc_blend)


def _mm_body(sel_ref, p_ref, x_ref, wb_ref, b0_ref, b1_ref, o_ref):
    b = pl.program_id(0)
    p0 = p_ref[b, 0]
    p1 = p_ref[b, 1]
    acc = jnp.dot(x_ref[0], wb_ref[0], preferred_element_type=jnp.float32)
    bias = p0 * b0_ref[0] + p1 * b1_ref[0]
    o_ref[0] = acc + bias


def kernel(x, selection_index, selection_probabilities, W_bank, b_bank):
    sel = selection_index.astype(jnp.int32)
    p = selection_probabilities.astype(jnp.float32)
    rows = jnp.arange(_IN_F, dtype=jnp.int32)[None, :]
    idx0 = (sel[:, 0:1] * _IN_F + rows).reshape(-1)
    idx1 = (sel[:, 1:2] * _IN_F + rows).reshape(-1)
    p0_rep = jnp.broadcast_to(p[:, 0:1], (_B, _L)).reshape(-1)
    p1_rep = jnp.broadcast_to(p[:, 1:2], (_B, _L)).reshape(-1)
    wflat = W_bank.reshape(_BANK * _IN_F, _OUT_F)
    b3 = b_bank.reshape(_BANK, 1, _OUT_F)

    wb = _blend_call(idx0, idx1, p0_rep, p1_rep, wflat)
    wb = wb.reshape(_B, _IN_F, _OUT_F)

    grid_spec = pltpu.PrefetchScalarGridSpec(
        num_scalar_prefetch=2,
        grid=(_B,),
        in_specs=[
            pl.BlockSpec((1, _S, _IN_F), lambda b, sel, p: (b, 0, 0)),
            pl.BlockSpec((1, _IN_F, _OUT_F), lambda b, sel, p: (b, 0, 0)),
            pl.BlockSpec((1, 1, _OUT_F), lambda b, sel, p: (sel[b, 0], 0, 0)),
            pl.BlockSpec((1, 1, _OUT_F), lambda b, sel, p: (sel[b, 1], 0, 0)),
        ],
        out_specs=pl.BlockSpec((1, _S, _OUT_F), lambda b, sel, p: (b, 0, 0)),
    )

    return pl.pallas_call(
        _mm_body,
        grid_spec=grid_spec,
        out_shape=jax.ShapeDtypeStruct((_B, _S, _OUT_F), jnp.float32),
    )(sel, p, x, wb, b3, b3)


# speedup vs baseline: 2.3276x; 2.3276x over previous
import functools

import jax
import jax.numpy as jnp
from jax.experimental import pallas as pl
from jax.experimental.pallas import tpu as pltpu

_B, _S, _IN_F, _OUT_F, _BANK, _K = 4, 2048, 1024, 1024, 16, 2
_S_TILE = 2048
_O_TILE = 1024


def _body(sel_ref, p_ref, x_ref, w0_ref, w1_ref, b0_ref, b1_ref, o_ref):
    b = pl.program_id(0)
    p0 = p_ref[b, 0]
    p1 = p_ref[b, 1]
    w = p0 * w0_ref[0] + p1 * w1_ref[0]
    acc = jnp.dot(x_ref[0], w, preferred_element_type=jnp.float32)
    bias = p0 * b0_ref[0] + p1 * b1_ref[0]
    o_ref[0] = acc + bias


def kernel(x, selection_index, selection_probabilities, W_bank, b_bank):
    sel = selection_index.astype(jnp.int32)
    p = selection_probabilities.astype(jnp.float32)
    b3 = b_bank.reshape(_BANK, 1, _OUT_F)
    grid = (_B, _OUT_F // _O_TILE, _S // _S_TILE)

    grid_spec = pltpu.PrefetchScalarGridSpec(
        num_scalar_prefetch=2,
        grid=grid,
        in_specs=[
            pl.BlockSpec((1, _S_TILE, _IN_F), lambda b, o, s, sel, p: (b, s, 0)),
            pl.BlockSpec((1, _IN_F, _O_TILE), lambda b, o, s, sel, p: (sel[b, 0], 0, o)),
            pl.BlockSpec((1, _IN_F, _O_TILE), lambda b, o, s, sel, p: (sel[b, 1], 0, o)),
            pl.BlockSpec((1, 1, _O_TILE), lambda b, o, s, sel, p: (sel[b, 0], 0, o)),
            pl.BlockSpec((1, 1, _O_TILE), lambda b, o, s, sel, p: (sel[b, 1], 0, o)),
        ],
        out_specs=pl.BlockSpec((1, _S_TILE, _O_TILE), lambda b, o, s, sel, p: (b, s, o)),
    )

    return pl.pallas_call(
        _body,
        grid_spec=grid_spec,
        out_shape=jax.ShapeDtypeStruct((_B, _S, _OUT_F), jnp.float32),
    )(sel, p, x, W_bank, W_bank, b3, b3)
